# baseline (speedup 1.0000x reference)
import functools

import jax
import jax.numpy as jnp
from jax import lax
from jax.experimental import pallas as pl
from jax.experimental.pallas import tpu as pltpu
from jax.experimental.pallas import tpu_sc as plsc

_NUM_TABLES = 4
_VOCAB = 1000000
_DIM = 32
_PER_KEY = 16384
_TOTAL = _NUM_TABLES * _PER_KEY

_info = plsc.get_sparse_core_info()
_NC, _NS = _info.num_cores, _info.num_subcores
_NW = _NC * _NS
_B_PER_W = _PER_KEY // _NW


def _grouped_gather(values, W0, W1, W2, W3):
    mesh = plsc.VectorSubcoreMesh(core_axis_name="c", subcore_axis_name="s")

    @functools.partial(
        pl.kernel,
        out_type=jax.ShapeDtypeStruct((_TOTAL, _DIM), jnp.float32),
        mesh=mesh,
        scratch_types=[
            pltpu.VMEM((_NUM_TABLES, _B_PER_W), jnp.int32),
            pltpu.VMEM((_NUM_TABLES, _B_PER_W, _DIM), jnp.float32),
            pltpu.SemaphoreType.DMA,
        ],
        compiler_params=pltpu.CompilerParams(use_tc_tiling_on_sc=False),
    )
    def k(values_hbm, w0_hbm, w1_hbm, w2_hbm, w3_hbm, out_hbm, idx_v, rows_v,
          sem):
        w = lax.axis_index("s") * _NC + lax.axis_index("c")
        tables = (w0_hbm, w1_hbm, w2_hbm, w3_hbm)
        copies = []
        for t, w_hbm in enumerate(tables):
            base = t * _PER_KEY + w * _B_PER_W
            pltpu.sync_copy(values_hbm.at[pl.ds(base, _B_PER_W)], idx_v.at[t])
            copies.append(
                pltpu.async_copy(w_hbm.at[idx_v.at[t]], rows_v.at[t], sem)
            )
        for t in range(_NUM_TABLES):
            copies[t].wait()
            base = t * _PER_KEY + w * _B_PER_W
            pltpu.sync_copy(rows_v.at[t], out_hbm.at[pl.ds(base, _B_PER_W)])

    return k(values, W0, W1, W2, W3)


@jax.jit
def kernel(values, W0, W1, W2, W3):
    return _grouped_gather(values, W0, W1, W2, W3)
